# baseline (speedup 1.0000x reference)
import functools

import jax
import jax.numpy as jnp
from jax import lax
from jax.experimental import pallas as pl
from jax.experimental.pallas import tpu as pltpu
from jax.experimental.pallas import tpu_sc as plsc

U = 10000
I = 10000
D = 128
HD = D // 2
NNZ = 320000

NC = 2
NS = 16

EB = 128
NBT = 160
NBLKP = NBT * NS
PAD = NBLKP * EB - NNZ
SB = 80
NRING = 4
ROWS_PER_TILE = 624


def _scale_block(gbuf, valsv, b):
    def group(g, carry):
        vv = valsv[b, pl.ds(g * 16, 16)]
        for l in range(16):
            v = vv[l]
            e = g * 16 + l
            for j in range(HD // 16):
                gbuf[e, pl.ds(j * 16, 16)] = gbuf[e, pl.ds(j * 16, 16)] * v
        return carry

    lax.fori_loop(0, EB // 16, group, 0)


def _zero_rows(buf):
    def zrow(r, carry):
        for j in range(HD // 16):
            buf[r, pl.ds(j * 16, 16)] = jnp.zeros((16,), jnp.float32)
        return carry

    lax.fori_loop(0, 128, zrow, 0)


def _spmm_accumulate(rows2, cols2, vals2, x_hbm, out_hbm,
                     acc, colsv, ridxv, valsv, gb, gs, ss, c, s):
    startblk = s * NBT
    rbase = s * ROWS_PER_TILE
    last_off = jnp.where(s == NS - 1, 512, 496)

    _zero_rows(gb[0])
    for off in (0, 128, 256, 384):
        pltpu.sync_copy(gb[0], acc.at[pl.ds(rbase + off, 128)])
    pltpu.sync_copy(gb[0], acc.at[pl.ds(rbase + last_off, 128)])
    plsc.subcore_barrier()

    for h in range(2):
        sb0 = startblk + h * SB
        pltpu.sync_copy(cols2.at[pl.ds(sb0, SB)], colsv)
        pltpu.sync_copy(rows2.at[pl.ds(sb0, SB)], ridxv)
        pltpu.sync_copy(vals2.at[pl.ds(sb0, SB)], valsv)

        def trow(i, carry):
            for j in range(EB // 16):
                v = colsv[i, pl.ds(j * 16, 16)]
                colsv[i, pl.ds(j * 16, 16)] = v + v + c
            return carry

        lax.fori_loop(0, SB, trow, 0)

        def process(b, j, first):
            nj = (j + NRING - 1) % NRING
            pltpu.make_async_copy(x_hbm.at[colsv.at[b]], gb[j], gs[j]).wait()
            _scale_block(gb[j], valsv, b)
            pltpu.async_copy(gb[j], acc.at[ridxv.at[b]], ss[j], add=True)

            @pl.when((b >= 1) & (b < SB - (NRING - 1)))
            def _refill():
                pltpu.make_async_copy(
                    gb[nj], acc.at[ridxv.at[b - 1]], ss[nj]).wait()
                pltpu.async_copy(
                    x_hbm.at[colsv.at[b + NRING - 1]], gb[nj], gs[nj])

            if first:
                @pl.when(b < 1)
                def _prime():
                    pltpu.async_copy(
                        x_hbm.at[colsv.at[NRING - 1]], gb[nj], gs[nj])

        for j in range(NRING - 1):
            pltpu.async_copy(x_hbm.at[colsv.at[j]], gb[j], gs[j])

        def ring(k, carry):
            for j in range(NRING):
                process(NRING * k + j, j, first=(j == 0))
            return carry

        lax.fori_loop(0, SB // NRING, ring, 0)
        for i in range(NRING):
            bb = SB - NRING + i
            pltpu.make_async_copy(gb[bb % NRING], acc.at[ridxv.at[bb]],
                                  ss[bb % NRING]).wait()

    plsc.subcore_barrier()

    iota2 = lax.iota(jnp.int32, 16) * 2

    def wchunk(off):
        base = 2 * (rbase + off) + c
        for j in range(EB // 16):
            ridxv[0, pl.ds(j * 16, 16)] = base + j * 32 + iota2
        pltpu.sync_copy(acc.at[pl.ds(rbase + off, 128)], gb[0])
        pltpu.sync_copy(gb[0], out_hbm.at[ridxv.at[0]])

    for off in (0, 128, 256, 384):
        wchunk(off)
    wchunk(last_off)
    plsc.subcore_barrier()


_SC_SCRATCH = [
    pltpu.VMEM_SHARED((U, HD), jnp.float32),
    pltpu.VMEM((SB, EB), jnp.int32),
    pltpu.VMEM((SB, EB), jnp.int32),
    pltpu.VMEM((SB, EB), jnp.float32),
] + [pltpu.VMEM((EB, HD), jnp.float32)] * NRING \
  + [pltpu.SemaphoreType.DMA] * (2 * NRING)

_MESH = plsc.VectorSubcoreMesh(core_axis_name="c", subcore_axis_name="s")
_SC_PARAMS = pltpu.CompilerParams(use_tc_tiling_on_sc=False)


@functools.partial(
    pl.kernel,
    out_type=[jax.ShapeDtypeStruct((2 * U, HD), jnp.float32)] * 4,
    mesh=_MESH,
    scratch_types=_SC_SCRATCH,
    compiler_params=_SC_PARAMS,
)
def _phase_a(eu, ei, ui_r, ui_c, ui_v, iu_r, iu_c, iu_v, u3_r, u3_c, u3_v,
             g1u, g1i, g3u, g3i,
             acc, colsv, ridxv, valsv, *bufs_and_sems):
    gb = bufs_and_sems[:NRING]
    gs = bufs_and_sems[NRING:2 * NRING]
    ss = bufs_and_sems[2 * NRING:]
    c = lax.axis_index("c")
    s = lax.axis_index("s")
    args = (acc, colsv, ridxv, valsv, gb, gs, ss, c, s)
    _spmm_accumulate(ui_r, ui_c, ui_v, ei, g1u, *args)
    _spmm_accumulate(iu_r, iu_c, iu_v, eu, g1i, *args)
    _spmm_accumulate(u3_r, u3_c, u3_v, ei, g3u, *args)
    _spmm_accumulate(u3_c, u3_r, u3_v, eu, g3i, *args)


@functools.partial(
    pl.kernel,
    out_type=[jax.ShapeDtypeStruct((2 * U, HD), jnp.float32)] * 2,
    mesh=_MESH,
    scratch_types=_SC_SCRATCH,
    compiler_params=_SC_PARAMS,
)
def _phase_b(g1u, g1i, ui_r, ui_c, ui_v, iu_r, iu_c, iu_v,
             g2u, g2i,
             acc, colsv, ridxv, valsv, *bufs_and_sems):
    gb = bufs_and_sems[:NRING]
    gs = bufs_and_sems[NRING:2 * NRING]
    ss = bufs_and_sems[2 * NRING:]
    c = lax.axis_index("c")
    s = lax.axis_index("s")
    args = (acc, colsv, ridxv, valsv, gb, gs, ss, c, s)
    _spmm_accumulate(ui_r, ui_c, ui_v, g1i, g2u, *args)
    _spmm_accumulate(iu_r, iu_c, iu_v, g1u, g2i, *args)


_RB = 1000
_GRID = U // _RB


def _combine_body(eu, ei, g1u, g1i, g2u, g2i, g3u, g3i, ujs, ou, oi):
    ou[...] = 0.25 * (eu[...] + g1u[...] + g2u[...]) + g3u[...] * ujs[...]
    oi[...] = 0.25 * (ei[...] + g1i[...] + g2i[...] + g3i[...])


def _combine(eu, ei, g1u, g1i, g2u, g2i, g3u, g3i, user_js):
    dense = pl.BlockSpec((_RB, D), lambda i: (i, 0))
    return pl.pallas_call(
        _combine_body,
        grid=(_GRID,),
        in_specs=[dense] * 8 + [pl.BlockSpec((_RB, 1), lambda i: (i, 0))],
        out_specs=[dense, dense],
        out_shape=[jax.ShapeDtypeStruct((U, D), jnp.float32)] * 2,
    )(eu, ei, g1u, g1i, g2u, g2i, g3u, g3i, user_js)


def kernel(embed_user, embed_item, ui_vals, iu_vals, ui3_vals, user_js,
           ui_rows, ui_cols, iu_rows, iu_cols, ui3_rows, ui3_cols):
    zi = jnp.zeros((PAD,), jnp.int32)
    zf = jnp.zeros((PAD,), jnp.float32)

    def blki(a):
        return jnp.concatenate([a.astype(jnp.int32), zi]).reshape(NBLKP, EB)

    def blkf(a):
        return jnp.concatenate([a, zf]).reshape(NBLKP, EB)

    def halves(x):
        return x.reshape(2 * U, HD)

    ui_r, ui_c, ui_v = blki(ui_rows), blki(ui_cols), blkf(ui_vals)
    iu_r, iu_c, iu_v = blki(iu_rows), blki(iu_cols), blkf(iu_vals)
    u3_r, u3_c, u3_v = blki(ui3_rows), blki(ui3_cols), blkf(ui3_vals)
    euh, eih = halves(embed_user), halves(embed_item)

    g1u, g1i, g3u, g3i = _phase_a(
        euh, eih, ui_r, ui_c, ui_v, iu_r, iu_c, iu_v, u3_r, u3_c, u3_v)
    g2u, g2i = _phase_b(g1u, g1i, ui_r, ui_c, ui_v, iu_r, iu_c, iu_v)

    def full(x):
        return x.reshape(U, D)

    return _combine(embed_user, embed_item, full(g1u), full(g1i),
                    full(g2u), full(g2i), full(g3u), full(g3i), user_js)
